# Initial kernel scaffold; baseline (speedup 1.0000x reference)
import jax
import jax.numpy as jnp
from jax.experimental import pallas as pl


def kernel(inputs, day_table, donor_table, cell_type_table, technology_table):
    raise NotImplementedError("write your pallas kernel here")


# speedup vs baseline: 4.9383x; 4.9383x over previous
import jax
import jax.numpy as jnp
from jax import lax
from jax.experimental import pallas as pl
from jax.experimental.pallas import tpu as pltpu
from jax.experimental.pallas import tpu_sc as plsc

_B = 16384
_NC = 2
_NS = 16
_NW = _NC * _NS
_ROWS_PER_W = _B // _NW
_CHUNK = 16
_NCHUNK = _ROWS_PER_W // _CHUNK


def _body(idx_hbm, day_hbm, donor_hbm, cell_hbm, tech_hbm, out_hbm,
          idx_v, day_v, donor_v, cell_v, tech_v, out_v):
    wid = lax.axis_index("s") * _NC + lax.axis_index("c")
    base = wid * _ROWS_PER_W
    pltpu.sync_copy(idx_hbm.at[pl.ds(base, _ROWS_PER_W)], idx_v)
    pltpu.sync_copy(day_hbm, day_v)
    pltpu.sync_copy(donor_hbm, donor_v)
    pltpu.sync_copy(cell_hbm, cell_v)
    pltpu.sync_copy(tech_hbm, tech_v)

    lane = lax.iota(jnp.int32, 16)
    tables = (day_v, donor_v, cell_v, tech_v)
    for j in range(_NCHUNK):
        rows = lane + (j * _CHUNK)
        for t in range(4):
            tcol = jnp.full((16,), t, jnp.int32)
            cidx = plsc.load_gather(idx_v, [rows, tcol])
            for c in range(2):
                ccol = jnp.full((16,), c, jnp.int32)
                vals = plsc.load_gather(tables[t], [cidx, ccol])
                ocol = jnp.full((16,), 2 * t + c, jnp.int32)
                plsc.store_scatter(out_v, [rows, ocol], vals)

    pltpu.sync_copy(out_v, out_hbm.at[pl.ds(base, _ROWS_PER_W)])


def kernel(inputs, day_table, donor_table, cell_type_table, technology_table):
    mesh = plsc.VectorSubcoreMesh(
        core_axis_name="c", subcore_axis_name="s",
        num_cores=_NC, num_subcores=_NS,
    )
    k = pl.kernel(
        _body,
        out_type=jax.ShapeDtypeStruct((_B, 8), jnp.float32),
        mesh=mesh,
        scratch_types=[
            pltpu.VMEM((_ROWS_PER_W, 4), jnp.int32),
            pltpu.VMEM((2, 2), jnp.float32),
            pltpu.VMEM((4, 2), jnp.float32),
            pltpu.VMEM((8, 2), jnp.float32),
            pltpu.VMEM((2, 2), jnp.float32),
            pltpu.VMEM((_ROWS_PER_W, 8), jnp.float32),
        ],
        compiler_params=pltpu.CompilerParams(
            needs_layout_passes=False, use_tc_tiling_on_sc=False),
    )
    return k(inputs, day_table, donor_table, cell_type_table, technology_table)


# speedup vs baseline: 5.0143x; 1.0154x over previous
import jax
import jax.numpy as jnp
from jax import lax
from jax.experimental import pallas as pl
from jax.experimental.pallas import tpu as pltpu
from jax.experimental.pallas import tpu_sc as plsc

_B = 16384
_NC = 2
_NS = 16
_NW = _NC * _NS
_ROWS_PER_W = _B // _NW
_CHUNK = 16
_NCHUNK = _ROWS_PER_W // _CHUNK


def _body(idx_hbm, day_hbm, donor_hbm, cell_hbm, tech_hbm, out_hbm,
          idx_v, day_v, donor_v, cell_v, tech_v, out_v):
    wid = lax.axis_index("s") * _NC + lax.axis_index("c")
    base = wid * _ROWS_PER_W
    pltpu.sync_copy(idx_hbm.at[pl.ds(base, _ROWS_PER_W)], idx_v)
    pltpu.sync_copy(day_hbm, day_v)
    pltpu.sync_copy(donor_hbm, donor_v)
    pltpu.sync_copy(cell_hbm, cell_v)
    pltpu.sync_copy(tech_hbm, tech_v)

    lane = lax.iota(jnp.int32, 16)
    tables = (day_v, donor_v, cell_v, tech_v)

    def chunk(j, _):
        rows = lane + j * _CHUNK
        for t in range(4):
            tcol = jnp.full((16,), t, jnp.int32)
            cidx = plsc.load_gather(idx_v, [rows, tcol])
            for c in range(2):
                ccol = jnp.full((16,), c, jnp.int32)
                vals = plsc.load_gather(tables[t], [cidx, ccol])
                ocol = jnp.full((16,), 2 * t + c, jnp.int32)
                plsc.store_scatter(out_v, [rows, ocol], vals)
        return _

    lax.fori_loop(0, _NCHUNK, chunk, 0)

    pltpu.sync_copy(out_v, out_hbm.at[pl.ds(base, _ROWS_PER_W)])


def kernel(inputs, day_table, donor_table, cell_type_table, technology_table):
    mesh = plsc.VectorSubcoreMesh(
        core_axis_name="c", subcore_axis_name="s",
        num_cores=_NC, num_subcores=_NS,
    )
    k = pl.kernel(
        _body,
        out_type=jax.ShapeDtypeStruct((_B, 8), jnp.float32),
        mesh=mesh,
        scratch_types=[
            pltpu.VMEM((_ROWS_PER_W, 4), jnp.int32),
            pltpu.VMEM((2, 2), jnp.float32),
            pltpu.VMEM((4, 2), jnp.float32),
            pltpu.VMEM((8, 2), jnp.float32),
            pltpu.VMEM((2, 2), jnp.float32),
            pltpu.VMEM((_ROWS_PER_W, 8), jnp.float32),
        ],
        compiler_params=pltpu.CompilerParams(
            needs_layout_passes=False, use_tc_tiling_on_sc=False),
    )
    return k(inputs, day_table, donor_table, cell_type_table, technology_table)


# speedup vs baseline: 11.3797x; 2.2695x over previous
import jax
import jax.numpy as jnp
from jax import lax
from jax.experimental import pallas as pl
from jax.experimental.pallas import tpu as pltpu
from jax.experimental.pallas import tpu_sc as plsc

_B = 16384
_NC = 2
_NS = 16
_NW = _NC * _NS
_NBLK = _B // 128
_BLK_PER_W = _NBLK // _NW
_NGRP = 128 // 16


def _body(idx_hbm, day_hbm, donor_hbm, cell_hbm, tech_hbm, out_hbm,
          idx_v, day_v, donor_v, cell_v, tech_v, out_v):
    wid = lax.axis_index("s") * _NC + lax.axis_index("c")
    base = wid * _BLK_PER_W
    pltpu.sync_copy(idx_hbm.at[pl.ds(base, _BLK_PER_W)], idx_v)
    pltpu.sync_copy(day_hbm, day_v)
    pltpu.sync_copy(donor_hbm, donor_v)
    pltpu.sync_copy(cell_hbm, cell_v)
    pltpu.sync_copy(tech_hbm, tech_v)

    tables = (day_v, donor_v, cell_v, tech_v)
    for b in range(_BLK_PER_W):
        for g in range(_NGRP):
            sl = pl.ds(g * 16, 16)
            for t in range(4):
                cidx = idx_v[b, t, sl]
                for c in range(2):
                    ccol = jnp.full((16,), c, jnp.int32)
                    out_v[b, 2 * t + c, sl] = plsc.load_gather(
                        tables[t], [cidx, ccol])

    pltpu.sync_copy(out_v, out_hbm.at[pl.ds(base, _BLK_PER_W)])


def kernel(inputs, day_table, donor_table, cell_type_table, technology_table):
    mesh = plsc.VectorSubcoreMesh(
        core_axis_name="c", subcore_axis_name="s",
        num_cores=_NC, num_subcores=_NS,
    )
    k = pl.kernel(
        _body,
        out_type=jax.ShapeDtypeStruct((_NBLK, 8, 128), jnp.float32),
        mesh=mesh,
        scratch_types=[
            pltpu.VMEM((_BLK_PER_W, 4, 128), jnp.int32),
            pltpu.VMEM((2, 2), jnp.float32),
            pltpu.VMEM((4, 2), jnp.float32),
            pltpu.VMEM((8, 2), jnp.float32),
            pltpu.VMEM((2, 2), jnp.float32),
            pltpu.VMEM((_BLK_PER_W, 8, 128), jnp.float32),
        ],
        compiler_params=pltpu.CompilerParams(
            needs_layout_passes=False, use_tc_tiling_on_sc=False),
    )
    idx3d = inputs.reshape(_NBLK, 128, 4).transpose(0, 2, 1)
    out3d = k(idx3d, day_table, donor_table, cell_type_table,
              technology_table)
    return out3d.transpose(0, 2, 1).reshape(_B, 8)


# speedup vs baseline: 12.2185x; 1.0737x over previous
import jax
import jax.numpy as jnp
from jax import lax
from jax.experimental import pallas as pl
from jax.experimental.pallas import tpu as pltpu
from jax.experimental.pallas import tpu_sc as plsc

_B = 16384
_NC = 2
_NS = 16
_NW = _NC * _NS
_NBLK = _B // 128
_BLK_PER_W = _NBLK // _NW
_NGRP = 128 // 16


def _body(idx_hbm, day_hbm, donor_hbm, cell_hbm, tech_hbm, out_hbm,
          idx_v, day_v, donor_v, cell_v, tech_v, out_v, sem):
    wid = lax.axis_index("s") * _NC + lax.axis_index("c")
    base = wid * _BLK_PER_W
    cps = [
        pltpu.async_copy(idx_hbm.at[pl.ds(base, _BLK_PER_W)], idx_v, sem),
        pltpu.async_copy(day_hbm, day_v, sem),
        pltpu.async_copy(donor_hbm, donor_v, sem),
        pltpu.async_copy(cell_hbm, cell_v, sem),
        pltpu.async_copy(tech_hbm, tech_v, sem),
    ]
    for cp in cps:
        cp.wait()

    tables = (day_v, donor_v, cell_v, tech_v)
    for b in range(_BLK_PER_W):
        for g in range(_NGRP):
            sl = pl.ds(g * 16, 16)
            for t in range(4):
                cidx = idx_v[b, t, sl]
                for c in range(2):
                    ccol = jnp.full((16,), c, jnp.int32)
                    out_v[b, 2 * t + c, sl] = plsc.load_gather(
                        tables[t], [cidx, ccol])

    pltpu.sync_copy(out_v, out_hbm.at[pl.ds(base, _BLK_PER_W)])


def kernel(inputs, day_table, donor_table, cell_type_table, technology_table):
    mesh = plsc.VectorSubcoreMesh(
        core_axis_name="c", subcore_axis_name="s",
        num_cores=_NC, num_subcores=_NS,
    )
    k = pl.kernel(
        _body,
        out_type=jax.ShapeDtypeStruct((_NBLK, 8, 128), jnp.float32),
        mesh=mesh,
        scratch_types=[
            pltpu.VMEM((_BLK_PER_W, 4, 128), jnp.int32),
            pltpu.VMEM((2, 2), jnp.float32),
            pltpu.VMEM((4, 2), jnp.float32),
            pltpu.VMEM((8, 2), jnp.float32),
            pltpu.VMEM((2, 2), jnp.float32),
            pltpu.VMEM((_BLK_PER_W, 8, 128), jnp.float32),
            pltpu.SemaphoreType.DMA,
        ],
        compiler_params=pltpu.CompilerParams(
            needs_layout_passes=False, use_tc_tiling_on_sc=False),
    )
    idx3d = inputs.reshape(_NBLK, 128, 4).transpose(0, 2, 1)
    out3d = k(idx3d, day_table, donor_table, cell_type_table,
              technology_table)
    return out3d.transpose(0, 2, 1).reshape(_B, 8)


# speedup vs baseline: 12.7320x; 1.0420x over previous
import jax
import jax.numpy as jnp
from jax import lax
from jax.experimental import pallas as pl
from jax.experimental.pallas import tpu as pltpu
from jax.experimental.pallas import tpu_sc as plsc

_B = 16384
_NC = 2
_NS = 16
_NW = _NC * _NS
_NBLK = _B // 128
_BLK_PER_W = _NBLK // _NW
_NGRP = 128 // 16
_TBASE = (0, 4, 12, 28)


def _body(idx_hbm, tbl_hbm, out_hbm, idx_v, tbl_v, out_v, sem):
    wid = lax.axis_index("s") * _NC + lax.axis_index("c")
    base = wid * _BLK_PER_W
    cps = [
        pltpu.async_copy(idx_hbm.at[pl.ds(base, _BLK_PER_W)], idx_v, sem),
        pltpu.async_copy(tbl_hbm, tbl_v, sem),
    ]
    for cp in cps:
        cp.wait()

    def group(g, _):
        sl = pl.ds(g * 16, 16)
        for b in range(_BLK_PER_W):
            for t in range(4):
                cidx2 = idx_v[b, t, sl] * 2 + _TBASE[t]
                for c in range(2):
                    out_v[b, 2 * t + c, sl] = plsc.load_gather(
                        tbl_v, [cidx2 + c])
        return _

    lax.fori_loop(0, _NGRP, group, 0)

    pltpu.sync_copy(out_v, out_hbm.at[pl.ds(base, _BLK_PER_W)])


def kernel(inputs, day_table, donor_table, cell_type_table, technology_table):
    mesh = plsc.VectorSubcoreMesh(
        core_axis_name="c", subcore_axis_name="s",
        num_cores=_NC, num_subcores=_NS,
    )
    k = pl.kernel(
        _body,
        out_type=jax.ShapeDtypeStruct((_NBLK, 8, 128), jnp.float32),
        mesh=mesh,
        scratch_types=[
            pltpu.VMEM((_BLK_PER_W, 4, 128), jnp.int32),
            pltpu.VMEM((32,), jnp.float32),
            pltpu.VMEM((_BLK_PER_W, 8, 128), jnp.float32),
            pltpu.SemaphoreType.DMA,
        ],
        compiler_params=pltpu.CompilerParams(
            needs_layout_passes=False, use_tc_tiling_on_sc=False),
    )
    idx3d = inputs.reshape(_NBLK, 128, 4).transpose(0, 2, 1)
    tbl = jnp.concatenate([
        day_table.reshape(-1), donor_table.reshape(-1),
        cell_type_table.reshape(-1), technology_table.reshape(-1),
    ])
    out3d = k(idx3d, tbl)
    return out3d.transpose(0, 2, 1).reshape(_B, 8)
